# baseline (speedup 1.0000x reference)
import functools

import jax
import jax.numpy as jnp
from jax import lax
from jax.experimental import pallas as pl
from jax.experimental.pallas import tpu as pltpu
from jax.experimental.pallas import tpu_sc as plsc

N = 10000
E = 320000
D = 128
H = 128

NC = 2
NS = 16
NW = NC * NS

B = 128
NB = 80
EPW = NB * B
EPAD = NW * EPW
TB = EPAD // B
BR = 2 * B

NBC = 40
NCH = 2

NPAD = 10240
RPT = NPAD // NS
ZR = B

_mesh = plsc.VectorSubcoreMesh(
    core_axis_name="c", subcore_axis_name="s", num_cores=NC, num_subcores=NS)


@functools.partial(
    pl.kernel,
    out_type=jax.ShapeDtypeStruct((NC, NPAD), jnp.float32),
    mesh=_mesh,
    scratch_types=[
        pltpu.VMEM((NB, B), jnp.int32),
        pltpu.VMEM((NB, B), jnp.float32),
        pltpu.VMEM((RPT,), jnp.float32),
        pltpu.VMEM_SHARED((NPAD,), jnp.float32),
    ],
)
def _sc_degree(dst_hbm, ew_hbm, out_hbm, dst_v, ew_v, buf_v, acc_sh):
    c = lax.axis_index("c")
    s = lax.axis_index("s")
    wid = c * NS + s

    def _zero(i, _):
        buf_v[pl.ds(i * 16, 16)] = jnp.zeros((16,), jnp.float32)
        return 0

    lax.fori_loop(0, RPT // 16, _zero, 0)
    pltpu.sync_copy(buf_v, acc_sh.at[pl.ds(s * RPT, RPT)])
    plsc.subcore_barrier()

    pltpu.sync_copy(dst_hbm.at[wid], dst_v)
    pltpu.sync_copy(ew_hbm.at[wid], ew_v)

    def _batch(j, _):
        pltpu.sync_copy(ew_v.at[j], acc_sh.at[dst_v.at[j]], add=True)
        return 0

    lax.fori_loop(0, NB, _batch, 0)
    plsc.subcore_barrier()

    pltpu.sync_copy(acc_sh.at[pl.ds(s * RPT, RPT)], buf_v)
    pltpu.sync_copy(buf_v, out_hbm.at[c, pl.ds(s * RPT, RPT)])


@functools.partial(
    pl.kernel,
    out_type=jax.ShapeDtypeStruct((NC, NPAD, H), jnp.float32),
    mesh=_mesh,
    scratch_types=[
        pltpu.VMEM((NBC, B), jnp.int32),
        pltpu.VMEM((NBC, B), jnp.int32),
        pltpu.VMEM((NBC, B), jnp.float32),
        pltpu.VMEM((BR, H), jnp.float32),
        pltpu.VMEM_SHARED((NPAD, H), jnp.float32),
        pltpu.SemaphoreType.DMA,
        pltpu.SemaphoreType.DMA,
    ],
)
def _sc_aggregate(y_hbm, src_hbm, dst_hbm, ew_hbm, out_hbm,
                  src_v, dst_v, ew_v, rows_v, acc_sh, sem0, sem1):
    c = lax.axis_index("c")
    s = lax.axis_index("s")

    def _zero(e, _):
        for q in range(H // 16):
            rows_v[e, pl.ds(q * 16, 16)] = jnp.zeros((16,), jnp.float32)
        return 0

    lax.fori_loop(0, ZR, _zero, 0)
    for k in range(RPT // ZR):
        pltpu.sync_copy(rows_v.at[pl.ds(0, ZR)],
                        acc_sh.at[pl.ds(s * RPT + k * ZR, ZR)])
    plsc.subcore_barrier()

    sems = (sem0, sem1)

    def _half(b):
        return rows_v.at[pl.ds(b * B, B)]

    def _scale(j, b):
        def _grp(g, _):
            wv = ew_v[j, pl.ds(g * 16, 16)]
            for k in range(16):
                w = wv[k]
                e = b * B + g * 16 + k
                for q in range(H // 16):
                    rows_v[e, pl.ds(q * 16, 16)] = (
                        rows_v[e, pl.ds(q * 16, 16)] * w)
            return 0

        lax.fori_loop(0, B // 16, _grp, 0)

    def _edges(nbc, nch, base):
        def _chunk(t, _):
            b0 = base + t * nbc
            pltpu.sync_copy(src_hbm.at[pl.ds(b0, nbc)],
                            src_v.at[pl.ds(0, nbc)])
            pltpu.sync_copy(dst_hbm.at[pl.ds(b0, nbc)],
                            dst_v.at[pl.ds(0, nbc)])
            pltpu.sync_copy(ew_hbm.at[pl.ds(b0, nbc)],
                            ew_v.at[pl.ds(0, nbc)])

            for b in range(2):
                pltpu.async_copy(y_hbm.at[src_v.at[b]], _half(b), sems[b])

            def _main(i, _):
                for b in range(2):
                    j = 2 * i + b
                    pltpu.make_async_copy(y_hbm.at[src_v.at[j]], _half(b),
                                          sems[b]).wait()
                    _scale(j, b)
                    pltpu.sync_copy(_half(b), acc_sh.at[dst_v.at[j]],
                                    add=True)
                    pltpu.async_copy(y_hbm.at[src_v.at[j + 2]], _half(b),
                                     sems[b])
                return 0

            lax.fori_loop(0, nbc // 2 - 1, _main, 0)
            for b in range(2):
                j = nbc - 2 + b
                pltpu.make_async_copy(y_hbm.at[src_v.at[j]], _half(b),
                                      sems[b]).wait()
                _scale(j, b)
                pltpu.sync_copy(_half(b), acc_sh.at[dst_v.at[j]], add=True)
            return 0

        lax.fori_loop(0, nch, _chunk, 0)

    _edges(NBC, NCH, (c * NS + s) * NB)

    plsc.subcore_barrier()

    for k in range(RPT // ZR):
        pltpu.sync_copy(acc_sh.at[pl.ds(s * RPT + k * ZR, ZR)],
                        rows_v.at[pl.ds(0, ZR)])
        pltpu.sync_copy(rows_v.at[pl.ds(0, ZR)],
                        out_hbm.at[c, pl.ds(s * RPT + k * ZR, ZR)])


BM = 2048


def _tc1_body(x_ref, h_ref, wg_ref, wixT_ref, whhT_ref, bih_ref, bhh_ref,
              xw_ref, gp_ref):
    xb = x_ref[...]
    xw_ref[...] = jnp.dot(xb, wg_ref[...], preferred_element_type=jnp.float32)
    gp_ref[...] = (
        jnp.dot(xb, wixT_ref[...], preferred_element_type=jnp.float32)
        + jnp.dot(h_ref[...], whhT_ref[...], preferred_element_type=jnp.float32)
        + bih_ref[...] + bhh_ref[...])


def _dis_chunk(degp_ref):
    deg = degp_ref[0, :] + degp_ref[1, :] + 1.0
    return jnp.where(deg > 0, lax.rsqrt(jnp.maximum(deg, 1e-12)), 0.0)


def _tc_scale_body(degp_ref, xw_ref, y_ref):
    dis = _dis_chunk(degp_ref)
    y_ref[...] = xw_ref[...] * dis[:, None]


def _tc2_body(aggp_ref, y_ref, degp_ref, gp_ref, c_ref, wgT_ref, bg_ref,
              h_out_ref, c_out_ref):
    dis = _dis_chunk(degp_ref)
    agg = aggp_ref[0] + aggp_ref[1]
    gnn = jax.nn.sigmoid(dis[:, None] * (agg + y_ref[...]) + bg_ref[...])
    gates = gp_ref[...] + jnp.dot(gnn, wgT_ref[...],
                                  preferred_element_type=jnp.float32)
    i = jax.nn.sigmoid(gates[:, 0 * H:1 * H])
    f = jax.nn.sigmoid(gates[:, 1 * H:2 * H])
    g = jnp.tanh(gates[:, 2 * H:3 * H])
    o = jax.nn.sigmoid(gates[:, 3 * H:4 * H])
    c_next = f * c_ref[...] + i * g
    h_out_ref[...] = o * jnp.tanh(c_next)
    c_out_ref[...] = c_next


def kernel(x, edge_index, edge_weight, h, c, W_gcn, b_gcn, W_ih, W_hh,
           b_ih, b_hh):
    src = edge_index[0]
    dst = edge_index[1]
    pad = EPAD - E
    src_r = jnp.pad(src, (0, pad)).reshape(NW, NB, B)
    dst_r = jnp.pad(dst, (0, pad), constant_values=N).reshape(NW, NB, B)
    ew_r = jnp.pad(edge_weight, (0, pad)).reshape(NW, NB, B)

    wixT = W_ih[:, :D].T
    wgT = W_ih[:, D:].T
    whhT = W_hh.T
    bih2 = b_ih.reshape(1, 4 * H)
    bhh2 = b_hh.reshape(1, 4 * H)
    bg2 = b_gcn.reshape(1, H)

    degp = _sc_degree(dst_r, ew_r)

    grid = pl.cdiv(N, BM)
    xw, gp = pl.pallas_call(
        _tc1_body,
        grid=(grid,),
        in_specs=[
            pl.BlockSpec((BM, D), lambda i: (i, 0)),
            pl.BlockSpec((BM, H), lambda i: (i, 0)),
            pl.BlockSpec((D, H), lambda i: (0, 0)),
            pl.BlockSpec((D, 4 * H), lambda i: (0, 0)),
            pl.BlockSpec((H, 4 * H), lambda i: (0, 0)),
            pl.BlockSpec((1, 4 * H), lambda i: (0, 0)),
            pl.BlockSpec((1, 4 * H), lambda i: (0, 0)),
        ],
        out_specs=[
            pl.BlockSpec((BM, H), lambda i: (i, 0)),
            pl.BlockSpec((BM, 4 * H), lambda i: (i, 0)),
        ],
        out_shape=[
            jax.ShapeDtypeStruct((N, H), jnp.float32),
            jax.ShapeDtypeStruct((N, 4 * H), jnp.float32),
        ],
    )(x, h, W_gcn, wixT, whhT, bih2, bhh2)

    y = pl.pallas_call(
        _tc_scale_body,
        grid=(grid,),
        in_specs=[
            pl.BlockSpec((NC, BM), lambda i: (0, i)),
            pl.BlockSpec((BM, H), lambda i: (i, 0)),
        ],
        out_specs=pl.BlockSpec((BM, H), lambda i: (i, 0)),
        out_shape=jax.ShapeDtypeStruct((N, H), jnp.float32),
    )(degp, xw)

    aggp = _sc_aggregate(y, src_r.reshape(TB, B), dst_r.reshape(TB, B),
                         ew_r.reshape(TB, B))

    h_next, c_next = pl.pallas_call(
        _tc2_body,
        grid=(grid,),
        in_specs=[
            pl.BlockSpec((NC, BM, H), lambda i: (0, i, 0)),
            pl.BlockSpec((BM, H), lambda i: (i, 0)),
            pl.BlockSpec((NC, BM), lambda i: (0, i)),
            pl.BlockSpec((BM, 4 * H), lambda i: (i, 0)),
            pl.BlockSpec((BM, H), lambda i: (i, 0)),
            pl.BlockSpec((H, 4 * H), lambda i: (0, 0)),
            pl.BlockSpec((1, H), lambda i: (0, 0)),
        ],
        out_specs=[
            pl.BlockSpec((BM, H), lambda i: (i, 0)),
            pl.BlockSpec((BM, H), lambda i: (i, 0)),
        ],
        out_shape=[
            jax.ShapeDtypeStruct((N, H), jnp.float32),
            jax.ShapeDtypeStruct((N, H), jnp.float32),
        ],
    )(aggp, y, degp, gp, c, wgT, bg2)

    return (h_next, c_next)


# speedup vs baseline: 1.3192x; 1.3192x over previous
import functools

import jax
import jax.numpy as jnp
from jax import lax
from jax.experimental import pallas as pl
from jax.experimental.pallas import tpu as pltpu
from jax.experimental.pallas import tpu_sc as plsc

N = 10000
E = 320000
D = 128
H = 128

NC = 2
NS = 16
NW = NC * NS

B = 128
NB = 80
EPW = NB * B
EPAD = NW * EPW
TB = EPAD // B
BR = 2 * B

NBC = 40
NCH = 2

NPAD = 10240
RPT = NPAD // NS
ZR = B

_mesh = plsc.VectorSubcoreMesh(
    core_axis_name="c", subcore_axis_name="s", num_cores=NC, num_subcores=NS)


@functools.partial(
    pl.kernel,
    out_type=jax.ShapeDtypeStruct((NC, NPAD), jnp.float32),
    mesh=_mesh,
    scratch_types=[
        pltpu.VMEM((NB, B), jnp.int32),
        pltpu.VMEM((NB, B), jnp.float32),
        pltpu.VMEM((RPT,), jnp.float32),
        pltpu.VMEM_SHARED((NPAD,), jnp.float32),
    ],
)
def _sc_degree(dst_hbm, ew_hbm, out_hbm, dst_v, ew_v, buf_v, acc_sh):
    c = lax.axis_index("c")
    s = lax.axis_index("s")
    wid = c * NS + s

    def _zero(i, _):
        buf_v[pl.ds(i * 16, 16)] = jnp.zeros((16,), jnp.float32)
        return 0

    lax.fori_loop(0, RPT // 16, _zero, 0)
    pltpu.sync_copy(buf_v, acc_sh.at[pl.ds(s * RPT, RPT)])
    plsc.subcore_barrier()

    pltpu.sync_copy(dst_hbm.at[wid], dst_v)
    pltpu.sync_copy(ew_hbm.at[wid], ew_v)

    def _batch(j, _):
        pltpu.sync_copy(ew_v.at[j], acc_sh.at[dst_v.at[j]], add=True)
        return 0

    lax.fori_loop(0, NB, _batch, 0)
    plsc.subcore_barrier()

    pltpu.sync_copy(acc_sh.at[pl.ds(s * RPT, RPT)], buf_v)
    pltpu.sync_copy(buf_v, out_hbm.at[c, pl.ds(s * RPT, RPT)])


@functools.partial(
    pl.kernel,
    out_type=jax.ShapeDtypeStruct((NC, NPAD, H), jnp.float32),
    mesh=_mesh,
    scratch_types=[
        pltpu.VMEM((NBC, B), jnp.int32),
        pltpu.VMEM((NBC, B), jnp.int32),
        pltpu.VMEM((NBC, B), jnp.float32),
        pltpu.VMEM((BR, H), jnp.float32),
        pltpu.VMEM_SHARED((NPAD, H), jnp.float32),
        pltpu.SemaphoreType.DMA,
        pltpu.SemaphoreType.DMA,
    ],
)
def _sc_aggregate(y_hbm, src_hbm, dst_hbm, ew_hbm, out_hbm,
                  src_v, dst_v, ew_v, rows_v, acc_sh, sem0, sem1):
    c = lax.axis_index("c")
    s = lax.axis_index("s")

    def _zero(e, _):
        for q in range(H // 16):
            rows_v[e, pl.ds(q * 16, 16)] = jnp.zeros((16,), jnp.float32)
        return 0

    lax.fori_loop(0, ZR, _zero, 0)
    for k in range(RPT // ZR):
        pltpu.sync_copy(rows_v.at[pl.ds(0, ZR)],
                        acc_sh.at[pl.ds(s * RPT + k * ZR, ZR)])
    plsc.subcore_barrier()

    sems = (sem0, sem1)

    def _half(b):
        return rows_v.at[pl.ds(b * B, B)]

    def _scale(j, b):
        def _grp(g, _):
            wv = ew_v[j, pl.ds(g * 16, 16)]
            for k in range(16):
                w = wv[k]
                e = b * B + g * 16 + k
                for q in range(H // 16):
                    rows_v[e, pl.ds(q * 16, 16)] = (
                        rows_v[e, pl.ds(q * 16, 16)] * w)
            return 0

        lax.fori_loop(0, B // 16, _grp, 0)

    wid = c * NS + s
    for t in range(NB // NBC):
        pltpu.sync_copy(src_hbm.at[wid, t], src_v)
        pltpu.sync_copy(dst_hbm.at[wid, t], dst_v)
        pltpu.sync_copy(ew_hbm.at[wid, t], ew_v)

        for b in range(2):
            pltpu.async_copy(y_hbm.at[src_v.at[b]], _half(b), sems[b])

        def _main(i, _):
            for b in range(2):
                j = 2 * i + b
                pltpu.make_async_copy(y_hbm.at[src_v.at[j]], _half(b),
                                      sems[b]).wait()
                _scale(j, b)
                pltpu.sync_copy(_half(b), acc_sh.at[dst_v.at[j]], add=True)
                pltpu.async_copy(y_hbm.at[src_v.at[j + 2]], _half(b), sems[b])
            return 0

        lax.fori_loop(0, NBC // 2 - 1, _main, 0)
        for b in range(2):
            j = NBC - 2 + b
            pltpu.make_async_copy(y_hbm.at[src_v.at[j]], _half(b),
                                  sems[b]).wait()
            _scale(j, b)
            pltpu.sync_copy(_half(b), acc_sh.at[dst_v.at[j]], add=True)

    plsc.subcore_barrier()

    for k in range(RPT // ZR):
        pltpu.sync_copy(acc_sh.at[pl.ds(s * RPT + k * ZR, ZR)],
                        rows_v.at[pl.ds(0, ZR)])
        pltpu.sync_copy(rows_v.at[pl.ds(0, ZR)],
                        out_hbm.at[c, pl.ds(s * RPT + k * ZR, ZR)])


BM = 2048


def _dis_chunk(degp_ref):
    deg = degp_ref[0, :] + degp_ref[1, :] + 1.0
    return jnp.where(deg > 0, lax.rsqrt(jnp.maximum(deg, 1e-12)), 0.0)


def _tc1_body(x_ref, h_ref, wg_ref, wixT_ref, whhT_ref, bih_ref, bhh_ref,
              xw_ref, gp_ref):
    xb = x_ref[...]
    xw_ref[...] = jnp.dot(xb, wg_ref[...], preferred_element_type=jnp.float32)
    gp_ref[...] = (
        jnp.dot(xb, wixT_ref[...], preferred_element_type=jnp.float32)
        + jnp.dot(h_ref[...], whhT_ref[...], preferred_element_type=jnp.float32)
        + bih_ref[...] + bhh_ref[...])


def _tc_scale_body(degp_ref, xw_ref, y_ref):
    dis = _dis_chunk(degp_ref)
    y_ref[...] = xw_ref[...] * dis[:, None]


def _tc2_body(aggp_ref, y_ref, degp_ref, gp_ref, c_ref, wgT_ref, bg_ref,
              h_out_ref, c_out_ref):
    dis = _dis_chunk(degp_ref)
    agg = aggp_ref[0] + aggp_ref[1]
    gnn = jax.nn.sigmoid(dis[:, None] * (agg + y_ref[...]) + bg_ref[...])
    gates = gp_ref[...] + jnp.dot(gnn, wgT_ref[...],
                                  preferred_element_type=jnp.float32)
    i = jax.nn.sigmoid(gates[:, 0 * H:1 * H])
    f = jax.nn.sigmoid(gates[:, 1 * H:2 * H])
    g = jnp.tanh(gates[:, 2 * H:3 * H])
    o = jax.nn.sigmoid(gates[:, 3 * H:4 * H])
    c_next = f * c_ref[...] + i * g
    h_out_ref[...] = o * jnp.tanh(c_next)
    c_out_ref[...] = c_next


def kernel(x, edge_index, edge_weight, h, c, W_gcn, b_gcn, W_ih, W_hh,
           b_ih, b_hh):
    src = edge_index[0]
    dst = edge_index[1]
    pad = EPAD - E
    src_r = jnp.pad(src, (0, pad)).reshape(NW, NB, B)
    dst_r = jnp.pad(dst, (0, pad), constant_values=N).reshape(NW, NB, B)
    ew_r = jnp.pad(edge_weight, (0, pad)).reshape(NW, NB, B)

    wixT = W_ih[:, :D].T
    wgT = W_ih[:, D:].T
    whhT = W_hh.T
    bih2 = b_ih.reshape(1, 4 * H)
    bhh2 = b_hh.reshape(1, 4 * H)
    bg2 = b_gcn.reshape(1, H)

    degp = _sc_degree(dst_r, ew_r)

    grid = pl.cdiv(N, BM)
    xw, gp = pl.pallas_call(
        _tc1_body,
        grid=(grid,),
        in_specs=[
            pl.BlockSpec((BM, D), lambda i: (i, 0)),
            pl.BlockSpec((BM, H), lambda i: (i, 0)),
            pl.BlockSpec((D, H), lambda i: (0, 0)),
            pl.BlockSpec((D, 4 * H), lambda i: (0, 0)),
            pl.BlockSpec((H, 4 * H), lambda i: (0, 0)),
            pl.BlockSpec((1, 4 * H), lambda i: (0, 0)),
            pl.BlockSpec((1, 4 * H), lambda i: (0, 0)),
        ],
        out_specs=[
            pl.BlockSpec((BM, H), lambda i: (i, 0)),
            pl.BlockSpec((BM, 4 * H), lambda i: (i, 0)),
        ],
        out_shape=[
            jax.ShapeDtypeStruct((N, H), jnp.float32),
            jax.ShapeDtypeStruct((N, 4 * H), jnp.float32),
        ],
    )(x, h, W_gcn, wixT, whhT, bih2, bhh2)

    y = pl.pallas_call(
        _tc_scale_body,
        grid=(grid,),
        in_specs=[
            pl.BlockSpec((NC, BM), lambda i: (0, i)),
            pl.BlockSpec((BM, H), lambda i: (i, 0)),
        ],
        out_specs=pl.BlockSpec((BM, H), lambda i: (i, 0)),
        out_shape=jax.ShapeDtypeStruct((N, H), jnp.float32),
    )(degp, xw)

    src_c = src_r.reshape(NW, NB // NBC, NBC, B)
    dst_c = dst_r.reshape(NW, NB // NBC, NBC, B)
    ew_c = ew_r.reshape(NW, NB // NBC, NBC, B)
    aggp = _sc_aggregate(y, src_c, dst_c, ew_c)

    h_next, c_next = pl.pallas_call(
        _tc2_body,
        grid=(grid,),
        in_specs=[
            pl.BlockSpec((NC, BM, H), lambda i: (0, i, 0)),
            pl.BlockSpec((BM, H), lambda i: (i, 0)),
            pl.BlockSpec((NC, BM), lambda i: (0, i)),
            pl.BlockSpec((BM, 4 * H), lambda i: (i, 0)),
            pl.BlockSpec((BM, H), lambda i: (i, 0)),
            pl.BlockSpec((H, 4 * H), lambda i: (0, 0)),
            pl.BlockSpec((1, H), lambda i: (0, 0)),
        ],
        out_specs=[
            pl.BlockSpec((BM, H), lambda i: (i, 0)),
            pl.BlockSpec((BM, H), lambda i: (i, 0)),
        ],
        out_shape=[
            jax.ShapeDtypeStruct((N, H), jnp.float32),
            jax.ShapeDtypeStruct((N, H), jnp.float32),
        ],
    )(aggp, y, degp, gp, c, wgT, bg2)

    return (h_next, c_next)
